# baseline (speedup 1.0000x reference)
import functools

import jax
import jax.numpy as jnp
from jax import lax
from jax.experimental import pallas as pl
from jax.experimental.pallas import tpu as pltpu
from jax.experimental.pallas import tpu_sc as plsc

V = 1000000
D = 64
B = 4096
H = 200
NW = 32
SCALE = 8.0
G = 4

_mesh = plsc.VectorSubcoreMesh(core_axis_name="c", subcore_axis_name="s")


def _iota16():
    return lax.iota(jnp.int32, 16)


@functools.partial(
    pl.kernel,
    out_type=jax.ShapeDtypeStruct((H, 8, 32, 8, 128), jnp.float32),
    mesh=_mesh,
    compiler_params=pltpu.CompilerParams(
        use_tc_tiling_on_sc=False, needs_layout_passes=False),
    scratch_types=[
        pltpu.VMEM((H, 128), jnp.int32),
        pltpu.VMEM((G, 128, D), jnp.float32),
        pltpu.VMEM((G, 8, 8, 136), jnp.float32),
        pltpu.SemaphoreType.DMA((G,)),
        pltpu.SemaphoreType.DMA((G,)),
    ],
)
def _hgather(xt_hbm, table_hbm, out_hbm, idx_v, gbuf, obuf, gsem, osem):
    wid = lax.axis_index("s") * 2 + lax.axis_index("c")
    pltpu.sync_copy(xt_hbm.at[:, pl.ds(wid * 128, 128)], idx_v)

    for g in range(G):
        pltpu.async_copy(table_hbm.at[idx_v.at[g]], gbuf.at[g], gsem.at[g])

    dts, dds = [], []
    for c in range(4):
        dvec = c * 16 + _iota16()
        dts.append(lax.shift_right_logical(dvec, 3))
        dds.append(lax.bitwise_and(dvec, 7))

    def outer(ii, carry):
        for g in range(G):
            h = ii * G + g
            pltpu.make_async_copy(
                table_hbm.at[idx_v.at[h]], gbuf.at[g], gsem.at[g]).wait()

            @pl.when(ii > 0)
            def _():
                pltpu.make_async_copy(
                    obuf.at[g, :, :, pl.ds(0, 128)],
                    out_hbm.at[0, pl.ds(0, 8), 0], osem.at[g]).wait()

            def rstep(r, c2):
                rv = jnp.broadcast_to(r, (16,))
                for c in range(4):
                    vals = gbuf[g, r, pl.ds(c * 16, 16)]
                    plsc.store_scatter(
                        obuf.at[g], [dts[c], dds[c], rv], vals * SCALE)
                return c2

            lax.fori_loop(0, 128, rstep, 0)

            pltpu.async_copy(
                obuf.at[g, :, :, pl.ds(0, 128)],
                out_hbm.at[h, pl.ds(0, 8), wid], osem.at[g])

            @pl.when(h + G < H)
            def _():
                pltpu.async_copy(
                    table_hbm.at[idx_v.at[h + G]], gbuf.at[g], gsem.at[g])
        return carry

    lax.fori_loop(0, H // G, outer, 0)

    for g in range(G):
        pltpu.make_async_copy(
            obuf.at[g, :, :, pl.ds(0, 128)],
            out_hbm.at[0, pl.ds(0, 8), 0], osem.at[g]).wait()


def kernel(x, table):
    xt = x.T
    op = _hgather(xt, table)
    return op.transpose(2, 4, 0, 1, 3).reshape(B, H, D)
